# baseline (speedup 1.0000x reference)
import jax
import jax.numpy as jnp
from jax.experimental import pallas as pl
from jax.experimental.pallas import tpu as pltpu

_LANES = 128
_ROWBLK = 512


def _qsample_body(t_ref, sa_ref, som_ref, x_ref, n_ref, o_ref):
    i = pl.program_id(0)
    tt = t_ref[i]
    o_ref[...] = sa_ref[tt] * x_ref[...] + som_ref[tt] * n_ref[...]


def kernel(x_start, t, noise, sqrt_alphas_cumprod, sqrt_one_minus_alphas_cumprod):
    b = x_start.shape[0]
    rows = x_start.size // (b * _LANES)
    x3 = x_start.reshape(b, rows, _LANES)
    n3 = noise.reshape(b, rows, _LANES)
    grid = (b, rows // _ROWBLK)
    spec = pl.BlockSpec((1, _ROWBLK, _LANES), lambda i, j, *_: (i, j, 0))
    out = pl.pallas_call(
        _qsample_body,
        grid_spec=pltpu.PrefetchScalarGridSpec(
            num_scalar_prefetch=3,
            grid=grid,
            in_specs=[spec, spec],
            out_specs=spec,
        ),
        out_shape=jax.ShapeDtypeStruct((b, rows, _LANES), jnp.float32),
    )(t, sqrt_alphas_cumprod, sqrt_one_minus_alphas_cumprod, x3, n3)
    return out.reshape(x_start.shape)


# speedup vs baseline: 1.2727x; 1.2727x over previous
import jax
import jax.numpy as jnp
from jax.experimental import pallas as pl
from jax.experimental.pallas import tpu as pltpu

_LANES = 128
_ROWBLK = 1536


def _qsample_body(t_ref, sa_ref, som_ref, x_ref, n_ref, o_ref):
    i = pl.program_id(0)
    tt = t_ref[i]
    o_ref[...] = sa_ref[tt] * x_ref[...] + som_ref[tt] * n_ref[...]


def kernel(x_start, t, noise, sqrt_alphas_cumprod, sqrt_one_minus_alphas_cumprod):
    b = x_start.shape[0]
    rows = x_start.size // (b * _LANES)
    x3 = x_start.reshape(b, rows, _LANES)
    n3 = noise.reshape(b, rows, _LANES)
    grid = (b, rows // _ROWBLK)
    spec = pl.BlockSpec((1, _ROWBLK, _LANES), lambda i, j, *_: (i, j, 0))
    out = pl.pallas_call(
        _qsample_body,
        grid_spec=pltpu.PrefetchScalarGridSpec(
            num_scalar_prefetch=3,
            grid=grid,
            in_specs=[spec, spec],
            out_specs=spec,
        ),
        out_shape=jax.ShapeDtypeStruct((b, rows, _LANES), jnp.float32),
    )(t, sqrt_alphas_cumprod, sqrt_one_minus_alphas_cumprod, x3, n3)
    return out.reshape(x_start.shape)


# speedup vs baseline: 1.3956x; 1.0966x over previous
import jax
import jax.numpy as jnp
from jax.experimental import pallas as pl
from jax.experimental.pallas import tpu as pltpu

_LANES = 128
_ROWS_PER_CHUNK = 1536
_NBUF = 4


def _qsample_body(t_ref, sa_ref, som_ref, x_hbm, n_hbm, o_hbm,
                  xb, nb, ob, xsem, nsem, osem):
    nchunks = t_ref.shape[0]

    def in_copies(c, slot):
        sl = pl.ds(c * _ROWS_PER_CHUNK, _ROWS_PER_CHUNK)
        cx = pltpu.make_async_copy(x_hbm.at[sl], xb.at[slot], xsem.at[slot])
        cn = pltpu.make_async_copy(n_hbm.at[sl], nb.at[slot], nsem.at[slot])
        return cx, cn

    def out_copy(c, slot):
        sl = pl.ds(c * _ROWS_PER_CHUNK, _ROWS_PER_CHUNK)
        return pltpu.make_async_copy(ob.at[slot], o_hbm.at[sl], osem.at[slot])

    for b in range(_NBUF):
        cx, cn = in_copies(b, b)
        cx.start()
        cn.start()

    for c in range(nchunks):
        slot = c % _NBUF
        cx, cn = in_copies(c, slot)
        cx.wait()
        cn.wait()
        if c >= _NBUF:
            out_copy(c - _NBUF, slot).wait()
        tt = t_ref[c]
        ob[slot] = sa_ref[tt] * xb[slot] + som_ref[tt] * nb[slot]
        out_copy(c, slot).start()
        nxt = c + _NBUF
        if nxt < nchunks:
            cx2, cn2 = in_copies(nxt, slot)
            cx2.start()
            cn2.start()

    for c in range(max(nchunks - _NBUF, 0), nchunks):
        out_copy(c, c % _NBUF).wait()


def kernel(x_start, t, noise, sqrt_alphas_cumprod, sqrt_one_minus_alphas_cumprod):
    b = x_start.shape[0]
    rows = x_start.size // (b * _LANES)
    assert rows == _ROWS_PER_CHUNK
    x2 = x_start.reshape(b * rows, _LANES)
    n2 = noise.reshape(b * rows, _LANES)
    smem = pl.BlockSpec(memory_space=pltpu.SMEM)
    hbm = pl.BlockSpec(memory_space=pltpu.MemorySpace.HBM)
    out = pl.pallas_call(
        _qsample_body,
        in_specs=[smem, smem, smem, hbm, hbm],
        out_specs=hbm,
        out_shape=jax.ShapeDtypeStruct((b * rows, _LANES), jnp.float32),
        scratch_shapes=[
            pltpu.VMEM((_NBUF, _ROWS_PER_CHUNK, _LANES), jnp.float32),
            pltpu.VMEM((_NBUF, _ROWS_PER_CHUNK, _LANES), jnp.float32),
            pltpu.VMEM((_NBUF, _ROWS_PER_CHUNK, _LANES), jnp.float32),
            pltpu.SemaphoreType.DMA((_NBUF,)),
            pltpu.SemaphoreType.DMA((_NBUF,)),
            pltpu.SemaphoreType.DMA((_NBUF,)),
        ],
    )(t.astype(jnp.int32), sqrt_alphas_cumprod, sqrt_one_minus_alphas_cumprod,
      x2, n2)
    return out.reshape(x_start.shape)


# speedup vs baseline: 1.4057x; 1.0072x over previous
import jax
import jax.numpy as jnp
from jax.experimental import pallas as pl
from jax.experimental.pallas import tpu as pltpu

_LANES = 128
_ROWS_PER_CHUNK = 1536
_NBUF = 8


def _qsample_body(t_ref, sa_ref, som_ref, x_hbm, n_hbm, o_hbm,
                  xb, nb, ob, xsem, nsem, osem):
    nchunks = t_ref.shape[0]

    def in_copies(c, slot):
        sl = pl.ds(c * _ROWS_PER_CHUNK, _ROWS_PER_CHUNK)
        cx = pltpu.make_async_copy(x_hbm.at[sl], xb.at[slot], xsem.at[slot])
        cn = pltpu.make_async_copy(n_hbm.at[sl], nb.at[slot], nsem.at[slot])
        return cx, cn

    def out_copy(c, slot):
        sl = pl.ds(c * _ROWS_PER_CHUNK, _ROWS_PER_CHUNK)
        return pltpu.make_async_copy(ob.at[slot], o_hbm.at[sl], osem.at[slot])

    for b in range(_NBUF):
        cx, cn = in_copies(b, b)
        cx.start()
        cn.start()

    for c in range(nchunks):
        slot = c % _NBUF
        cx, cn = in_copies(c, slot)
        cx.wait()
        cn.wait()
        if c >= _NBUF:
            out_copy(c - _NBUF, slot).wait()
        tt = t_ref[c]
        ob[slot] = sa_ref[tt] * xb[slot] + som_ref[tt] * nb[slot]
        out_copy(c, slot).start()
        nxt = c + _NBUF
        if nxt < nchunks:
            cx2, cn2 = in_copies(nxt, slot)
            cx2.start()
            cn2.start()

    for c in range(max(nchunks - _NBUF, 0), nchunks):
        out_copy(c, c % _NBUF).wait()


def kernel(x_start, t, noise, sqrt_alphas_cumprod, sqrt_one_minus_alphas_cumprod):
    b = x_start.shape[0]
    rows = x_start.size // (b * _LANES)
    assert rows == _ROWS_PER_CHUNK
    x2 = x_start.reshape(b * rows, _LANES)
    n2 = noise.reshape(b * rows, _LANES)
    smem = pl.BlockSpec(memory_space=pltpu.SMEM)
    hbm = pl.BlockSpec(memory_space=pltpu.MemorySpace.HBM)
    out = pl.pallas_call(
        _qsample_body,
        in_specs=[smem, smem, smem, hbm, hbm],
        out_specs=hbm,
        out_shape=jax.ShapeDtypeStruct((b * rows, _LANES), jnp.float32),
        scratch_shapes=[
            pltpu.VMEM((_NBUF, _ROWS_PER_CHUNK, _LANES), jnp.float32),
            pltpu.VMEM((_NBUF, _ROWS_PER_CHUNK, _LANES), jnp.float32),
            pltpu.VMEM((_NBUF, _ROWS_PER_CHUNK, _LANES), jnp.float32),
            pltpu.SemaphoreType.DMA((_NBUF,)),
            pltpu.SemaphoreType.DMA((_NBUF,)),
            pltpu.SemaphoreType.DMA((_NBUF,)),
        ],
    )(t.astype(jnp.int32), sqrt_alphas_cumprod, sqrt_one_minus_alphas_cumprod,
      x2, n2)
    return out.reshape(x_start.shape)


# speedup vs baseline: 5.8717x; 4.1772x over previous
import jax
import jax.numpy as jnp
from jax.experimental import pallas as pl
from jax.experimental.pallas import tpu as pltpu

_NBUF = 4


def _qsample_body(t_ref, sa_ref, som_ref, x_hbm, n_hbm, o_hbm,
                  xb, nb, ob, xsem, nsem, osem):
    nchunks = t_ref.shape[0]

    def in_copies(c, slot):
        cx = pltpu.make_async_copy(x_hbm.at[c], xb.at[slot], xsem.at[slot])
        cn = pltpu.make_async_copy(n_hbm.at[c], nb.at[slot], nsem.at[slot])
        return cx, cn

    def out_copy(c, slot):
        return pltpu.make_async_copy(ob.at[slot], o_hbm.at[c], osem.at[slot])

    for b in range(_NBUF):
        cx, cn = in_copies(b, b)
        cx.start()
        cn.start()

    for c in range(nchunks):
        slot = c % _NBUF
        cx, cn = in_copies(c, slot)
        cx.wait()
        cn.wait()
        if c >= _NBUF:
            out_copy(c - _NBUF, slot).wait()
        tt = t_ref[c]
        ob[slot] = sa_ref[tt] * xb[slot] + som_ref[tt] * nb[slot]
        out_copy(c, slot).start()
        nxt = c + _NBUF
        if nxt < nchunks:
            cx2, cn2 = in_copies(nxt, slot)
            cx2.start()
            cn2.start()

    for c in range(max(nchunks - _NBUF, 0), nchunks):
        out_copy(c, c % _NBUF).wait()


def kernel(x_start, t, noise, sqrt_alphas_cumprod, sqrt_one_minus_alphas_cumprod):
    b, ch, h, w = x_start.shape
    smem = pl.BlockSpec(memory_space=pltpu.SMEM)
    hbm = pl.BlockSpec(memory_space=pltpu.MemorySpace.HBM)
    buf = pltpu.VMEM((_NBUF, ch, h, w), jnp.float32)
    return pl.pallas_call(
        _qsample_body,
        in_specs=[smem, smem, smem, hbm, hbm],
        out_specs=hbm,
        out_shape=jax.ShapeDtypeStruct((b, ch, h, w), jnp.float32),
        scratch_shapes=[
            buf, buf, buf,
            pltpu.SemaphoreType.DMA((_NBUF,)),
            pltpu.SemaphoreType.DMA((_NBUF,)),
            pltpu.SemaphoreType.DMA((_NBUF,)),
        ],
    )(t.astype(jnp.int32), sqrt_alphas_cumprod, sqrt_one_minus_alphas_cumprod,
      x_start, noise)


# speedup vs baseline: 5.9240x; 1.0089x over previous
import jax
import jax.numpy as jnp
from jax.experimental import pallas as pl
from jax.experimental.pallas import tpu as pltpu

_NBUF = 6


def _qsample_body(t_ref, sa_ref, som_ref, x_hbm, n_hbm, o_hbm,
                  xb, nb, ob, xsem, nsem, osem):
    nchunks = t_ref.shape[0]

    def in_copies(c, slot):
        cx = pltpu.make_async_copy(x_hbm.at[c], xb.at[slot], xsem.at[slot])
        cn = pltpu.make_async_copy(n_hbm.at[c], nb.at[slot], nsem.at[slot])
        return cx, cn

    def out_copy(c, slot):
        return pltpu.make_async_copy(ob.at[slot], o_hbm.at[c], osem.at[slot])

    for b in range(_NBUF):
        cx, cn = in_copies(b, b)
        cx.start()
        cn.start()

    for c in range(nchunks):
        slot = c % _NBUF
        cx, cn = in_copies(c, slot)
        cx.wait()
        cn.wait()
        if c >= _NBUF:
            out_copy(c - _NBUF, slot).wait()
        tt = t_ref[c]
        ob[slot] = sa_ref[tt] * xb[slot] + som_ref[tt] * nb[slot]
        out_copy(c, slot).start()
        nxt = c + _NBUF
        if nxt < nchunks:
            cx2, cn2 = in_copies(nxt, slot)
            cx2.start()
            cn2.start()

    for c in range(max(nchunks - _NBUF, 0), nchunks):
        out_copy(c, c % _NBUF).wait()


def kernel(x_start, t, noise, sqrt_alphas_cumprod, sqrt_one_minus_alphas_cumprod):
    b, ch, h, w = x_start.shape
    smem = pl.BlockSpec(memory_space=pltpu.SMEM)
    hbm = pl.BlockSpec(memory_space=pltpu.MemorySpace.HBM)
    buf = pltpu.VMEM((_NBUF, ch, h, w), jnp.float32)
    return pl.pallas_call(
        _qsample_body,
        in_specs=[smem, smem, smem, hbm, hbm],
        out_specs=hbm,
        out_shape=jax.ShapeDtypeStruct((b, ch, h, w), jnp.float32),
        scratch_shapes=[
            buf, buf, buf,
            pltpu.SemaphoreType.DMA((_NBUF,)),
            pltpu.SemaphoreType.DMA((_NBUF,)),
            pltpu.SemaphoreType.DMA((_NBUF,)),
        ],
    )(t.astype(jnp.int32), sqrt_alphas_cumprod, sqrt_one_minus_alphas_cumprod,
      x_start, noise)
